# baseline (speedup 1.0000x reference)
import functools

import jax
import jax.numpy as jnp
from jax import lax
from jax.experimental import pallas as pl
from jax.experimental.pallas import tpu as pltpu
from jax.experimental.pallas import tpu_sc as plsc

NC = 2
NS = 16
NW = NC * NS
CH = 64


def _mesh():
    return plsc.VectorSubcoreMesh(core_axis_name="c", subcore_axis_name="s")


def _sc_gather(mem, idx3, nch):
    d = mem.shape[1]
    b = NW * nch * CH

    @functools.partial(
        pl.kernel,
        mesh=_mesh(),
        out_type=jax.ShapeDtypeStruct((b, d), jnp.float32),
        scratch_types=[
            pltpu.VMEM((nch, CH), jnp.int32),
            pltpu.VMEM((CH, d), jnp.float32),
            pltpu.VMEM((CH, d), jnp.float32),
            pltpu.SemaphoreType.DMA,
            pltpu.SemaphoreType.DMA,
            pltpu.SemaphoreType.DMA,
            pltpu.SemaphoreType.DMA,
        ],
    )
    def k(mem_hbm, idx_hbm, out_hbm, idx_v, rows0, rows1, g0, g1, w0, w1):
        wid = lax.axis_index("s") * NC + lax.axis_index("c")
        base = wid * nch * CH
        pltpu.sync_copy(idx_hbm.at[wid], idx_v)
        bufs = (rows0, rows1)
        gsem = (g0, g1)
        wsem = (w0, w1)

        def wb_slice(j):
            return out_hbm.at[pl.ds(base + j * CH, CH)]

        pltpu.async_copy(mem_hbm.at[idx_v.at[0]], bufs[0], gsem[0])
        for j in range(nch):
            s = j % 2
            if j + 1 < nch:
                ns = (j + 1) % 2
                if j >= 1:
                    pltpu.make_async_copy(
                        bufs[ns], wb_slice(j - 1), wsem[ns]).wait()
                pltpu.async_copy(
                    mem_hbm.at[idx_v.at[j + 1]], bufs[ns], gsem[ns])
            pltpu.make_async_copy(
                mem_hbm.at[idx_v.at[j]], bufs[s], gsem[s]).wait()
            pltpu.async_copy(bufs[s], wb_slice(j), wsem[s])
        pltpu.make_async_copy(
            bufs[(nch - 1) % 2], wb_slice(nch - 1), wsem[(nch - 1) % 2]).wait()
        if nch >= 2:
            pltpu.make_async_copy(
                bufs[(nch - 2) % 2], wb_slice(nch - 2),
                wsem[(nch - 2) % 2]).wait()

    return k(mem, idx3)


def _sc_scatter(mem_ref, last_ref, new_mem, ts, dst3, src3, dstf, srcf, nch):
    d = new_mem.shape[1]
    ntc = dstf.shape[1]

    @functools.partial(
        pl.kernel,
        mesh=_mesh(),
        out_type=(),
        scratch_types=[
            pltpu.VMEM((nch, CH), jnp.int32),
            pltpu.VMEM((nch, CH), jnp.int32),
            pltpu.VMEM((ntc, 128), jnp.int32),
            pltpu.VMEM((ntc, 128), jnp.int32),
            pltpu.VMEM((CH, d), jnp.float32),
            pltpu.VMEM((CH, d), jnp.float32),
            pltpu.VMEM((ntc, 128), jnp.float32),
            pltpu.SemaphoreType.DMA,
            pltpu.SemaphoreType.DMA,
            pltpu.SemaphoreType.DMA,
            pltpu.SemaphoreType.DMA,
            pltpu.SemaphoreType.DMA,
        ],
    )
    def k(new_hbm, ts_hbm, dst_hbm, src_hbm, dstf_hbm, srcf_hbm,
          mem_out, last_out,
          dst_v, src_v, dstf_v, srcf_v, rows0, rows1, ts_v,
          g0, g1, s0, s1, semt):
        wid = lax.axis_index("s") * NC + lax.axis_index("c")
        pltpu.sync_copy(dst_hbm.at[wid], dst_v)
        pltpu.sync_copy(src_hbm.at[wid], src_v)
        pltpu.sync_copy(dstf_hbm.at[wid], dstf_v)
        pltpu.sync_copy(srcf_hbm.at[wid], srcf_v)

        for j in range(ntc):
            pltpu.async_copy(ts_hbm.at[srcf_v.at[j]], ts_v.at[j], semt)

        bufs = (rows0, rows1)
        gsem = (g0, g1)
        ssem = (s0, s1)
        pltpu.async_copy(new_hbm.at[src_v.at[0]], bufs[0], gsem[0])
        for j in range(nch):
            s = j % 2
            if j + 1 < nch:
                ns = (j + 1) % 2
                if j >= 1:
                    pltpu.make_async_copy(
                        bufs[ns], mem_out.at[dst_v.at[j - 1]], ssem[ns]).wait()
                pltpu.async_copy(
                    new_hbm.at[src_v.at[j + 1]], bufs[ns], gsem[ns])
            pltpu.make_async_copy(
                new_hbm.at[src_v.at[j]], bufs[s], gsem[s]).wait()
            pltpu.async_copy(bufs[s], mem_out.at[dst_v.at[j]], ssem[s])

        for j in range(ntc):
            pltpu.make_async_copy(
                ts_hbm.at[srcf_v.at[j]], ts_v.at[j], semt).wait()
        for j in range(ntc):
            pltpu.async_copy(ts_v.at[j], last_out.at[dstf_v.at[j]], semt)
        for j in range(ntc):
            pltpu.make_async_copy(
                ts_v.at[j], last_out.at[dstf_v.at[j]], semt).wait()

        pltpu.make_async_copy(
            bufs[(nch - 1) % 2], mem_out.at[dst_v.at[nch - 1]],
            ssem[(nch - 1) % 2]).wait()
        if nch >= 2:
            pltpu.make_async_copy(
                bufs[(nch - 2) % 2], mem_out.at[dst_v.at[nch - 2]],
                ssem[(nch - 2) % 2]).wait()

    k(new_mem, ts, dst3, src3, dstf, srcf, mem_ref, last_ref)


def _tc_copy(mem, last2d):
    u, d = mem.shape
    rows = 1000
    grid = u // rows
    lrows = last2d.shape[0] // grid
    lcols = last2d.shape[1]

    def body(m_r, l_r, mo_r, lo_r):
        mo_r[:] = m_r[:]
        lo_r[:] = l_r[:]

    return pl.pallas_call(
        body,
        grid=(grid,),
        in_specs=[
            pl.BlockSpec((rows, d), lambda i: (i, 0)),
            pl.BlockSpec((lrows, lcols), lambda i: (i, 0)),
        ],
        out_specs=[
            pl.BlockSpec((rows, d), lambda i: (i, 0)),
            pl.BlockSpec((lrows, lcols), lambda i: (i, 0)),
        ],
        out_shape=[
            jax.ShapeDtypeStruct((u, d), jnp.float32),
            jax.ShapeDtypeStruct(last2d.shape, jnp.float32),
        ],
        compiler_params=pltpu.CompilerParams(
            dimension_semantics=("arbitrary",)),
    )(mem, last2d)


def _dense(um, ie, ft, w1, b1, w2, b2, w_ih, w_hh, b_ih, b_hh):
    b, d = um.shape
    msg = w1.shape[0]
    tb = 512
    grid = b // tb

    bf16 = jnp.bfloat16
    w1t = w1.T.astype(bf16)
    w1a, w1b, w1c = w1t[:d], w1t[d:2 * d], w1t[2 * d:]
    w2t = w2.T.astype(bf16)
    wih = w_ih.T.astype(bf16)
    whh = w_hh.T.astype(bf16)
    b1r = b1.reshape(1, -1)
    b2r = b2.reshape(1, -1)
    bihr = b_ih.reshape(1, -1)
    bhhr = b_hh.reshape(1, -1)

    def body(um_r, ie_r, ft_r, w1a_r, w1b_r, w1c_r, w2t_r, wih_r, whh_r,
             b1_r, b2_r, bih_r, bhh_r, out_r):
        umv = um_r[:]
        f32 = jnp.float32
        bf = jnp.bfloat16
        umb = umv.astype(bf)
        x = jnp.dot(umb, w1a_r[:], preferred_element_type=f32)
        x = x + jnp.dot(ie_r[:].astype(bf), w1b_r[:], preferred_element_type=f32)
        x = x + jnp.dot(ft_r[:].astype(bf), w1c_r[:], preferred_element_type=f32)
        h1 = jnp.maximum(x + b1_r[:], 0.0)
        m = jnp.dot(h1.astype(bf), w2t_r[:], preferred_element_type=f32) + b2_r[:]
        gi = jnp.dot(m.astype(bf), wih_r[:], preferred_element_type=f32) + bih_r[:]
        gh = jnp.dot(umb, whh_r[:], preferred_element_type=f32) + bhh_r[:]
        r = jax.nn.sigmoid(gi[:, :d] + gh[:, :d])
        z = jax.nn.sigmoid(gi[:, d:2 * d] + gh[:, d:2 * d])
        n = jnp.tanh(gi[:, 2 * d:] + r * gh[:, 2 * d:])
        out_r[:] = (1.0 - z) * n + z * umv

    const = lambda shape: pl.BlockSpec(shape, lambda i: (0, 0))
    batch = lambda shape: pl.BlockSpec(shape, lambda i: (i, 0))
    return pl.pallas_call(
        body,
        grid=(grid,),
        in_specs=[
            batch((tb, d)), batch((tb, d)), batch((tb, msg)),
            const((d, msg)), const((d, msg)), const((msg, msg)),
            const((msg, msg)), const((msg, 3 * d)), const((d, 3 * d)),
            const((1, msg)), const((1, msg)),
            const((1, 3 * d)), const((1, 3 * d)),
        ],
        out_specs=batch((tb, d)),
        out_shape=jax.ShapeDtypeStruct((b, d), jnp.float32),
        compiler_params=pltpu.CompilerParams(
            dimension_semantics=("arbitrary",)),
    )(um, ie, ft, w1a, w1b, w1c, w2t, wih, whh, b1r, b2r, bihr, bhhr)


def kernel(memory, last_update, user_indices, item_embedding,
           interaction_features, timestamps,
           w1, b1, w2, b2, w_ih, w_hh, b_ih, b_hh):
    u = memory.shape[0]
    b = user_indices.shape[0]
    nch = b // (NW * CH)

    ui = user_indices.astype(jnp.int32)
    iota = jnp.arange(b, dtype=jnp.int32)
    win = jnp.zeros((u,), jnp.uint16).at[ui].set(iota.astype(jnp.uint16))
    src = win[ui].astype(jnp.int32)

    idx3 = ui.reshape(NW, nch, CH)
    src3 = src.reshape(NW, nch, CH)
    ntc = b // (NW * 128)
    idxf = ui.reshape(NW, ntc, 128)
    srcf = src.reshape(NW, ntc, 128)

    user_memory = _sc_gather(memory, idx3, nch)
    new_memory = _dense(user_memory, item_embedding, interaction_features,
                        w1, b1, w2, b2, w_ih, w_hh, b_ih, b_hh)

    mem_ref = jax.new_ref(memory)
    last_ref = jax.new_ref(last_update)
    _sc_scatter(mem_ref, last_ref, new_memory, timestamps,
                idx3, src3, idxf, srcf, nch)
    return new_memory, mem_ref[...], last_ref[...]


# speedup vs baseline: 1.0379x; 1.0379x over previous
import functools

import jax
import jax.numpy as jnp
from jax import lax
from jax.experimental import pallas as pl
from jax.experimental.pallas import tpu as pltpu
from jax.experimental.pallas import tpu_sc as plsc

NC = 2
NS = 16
NW = NC * NS
CH = 64


def _mesh():
    return plsc.VectorSubcoreMesh(core_axis_name="c", subcore_axis_name="s")


def _sc_gather(mem, idx3, nch):
    d = mem.shape[1]
    b = NW * nch * CH

    @functools.partial(
        pl.kernel,
        mesh=_mesh(),
        out_type=jax.ShapeDtypeStruct((b, d), jnp.float32),
        scratch_types=[
            pltpu.VMEM((nch, CH), jnp.int32),
            pltpu.VMEM((CH, d), jnp.float32),
            pltpu.VMEM((CH, d), jnp.float32),
            pltpu.SemaphoreType.DMA,
            pltpu.SemaphoreType.DMA,
            pltpu.SemaphoreType.DMA,
            pltpu.SemaphoreType.DMA,
        ],
    )
    def k(mem_hbm, idx_hbm, out_hbm, idx_v, rows0, rows1, g0, g1, w0, w1):
        wid = lax.axis_index("s") * NC + lax.axis_index("c")
        base = wid * nch * CH
        pltpu.sync_copy(idx_hbm.at[wid], idx_v)
        bufs = (rows0, rows1)
        gsem = (g0, g1)
        wsem = (w0, w1)

        def wb_slice(j):
            return out_hbm.at[pl.ds(base + j * CH, CH)]

        pltpu.async_copy(mem_hbm.at[idx_v.at[0]], bufs[0], gsem[0])
        for j in range(nch):
            s = j % 2
            if j + 1 < nch:
                ns = (j + 1) % 2
                if j >= 1:
                    pltpu.make_async_copy(
                        bufs[ns], wb_slice(j - 1), wsem[ns]).wait()
                pltpu.async_copy(
                    mem_hbm.at[idx_v.at[j + 1]], bufs[ns], gsem[ns])
            pltpu.make_async_copy(
                mem_hbm.at[idx_v.at[j]], bufs[s], gsem[s]).wait()
            pltpu.async_copy(bufs[s], wb_slice(j), wsem[s])
        pltpu.make_async_copy(
            bufs[(nch - 1) % 2], wb_slice(nch - 1), wsem[(nch - 1) % 2]).wait()
        if nch >= 2:
            pltpu.make_async_copy(
                bufs[(nch - 2) % 2], wb_slice(nch - 2),
                wsem[(nch - 2) % 2]).wait()

    return k(mem, idx3)


def _sc_scatter(mem_ref, last_ref, new_mem, ts, dst3, src3, dstf, srcf, nch):
    d = new_mem.shape[1]
    ntc = dstf.shape[1]

    @functools.partial(
        pl.kernel,
        mesh=_mesh(),
        out_type=(),
        scratch_types=[
            pltpu.VMEM((nch, CH), jnp.int32),
            pltpu.VMEM((nch, CH), jnp.int32),
            pltpu.VMEM((ntc, 128), jnp.int32),
            pltpu.VMEM((ntc, 128), jnp.int32),
            pltpu.VMEM((CH, d), jnp.float32),
            pltpu.VMEM((CH, d), jnp.float32),
            pltpu.VMEM((ntc, 128), jnp.float32),
            pltpu.SemaphoreType.DMA,
            pltpu.SemaphoreType.DMA,
            pltpu.SemaphoreType.DMA,
            pltpu.SemaphoreType.DMA,
            pltpu.SemaphoreType.DMA,
        ],
    )
    def k(new_hbm, ts_hbm, dst_hbm, src_hbm, dstf_hbm, srcf_hbm,
          mem_out, last_out,
          dst_v, src_v, dstf_v, srcf_v, rows0, rows1, ts_v,
          g0, g1, s0, s1, semt):
        wid = lax.axis_index("s") * NC + lax.axis_index("c")
        pltpu.sync_copy(dst_hbm.at[wid], dst_v)
        pltpu.sync_copy(src_hbm.at[wid], src_v)
        pltpu.sync_copy(dstf_hbm.at[wid], dstf_v)
        pltpu.sync_copy(srcf_hbm.at[wid], srcf_v)

        for j in range(ntc):
            pltpu.async_copy(ts_hbm.at[srcf_v.at[j]], ts_v.at[j], semt)

        bufs = (rows0, rows1)
        gsem = (g0, g1)
        ssem = (s0, s1)
        pltpu.async_copy(new_hbm.at[src_v.at[0]], bufs[0], gsem[0])
        for j in range(nch):
            s = j % 2
            if j + 1 < nch:
                ns = (j + 1) % 2
                if j >= 1:
                    pltpu.make_async_copy(
                        bufs[ns], mem_out.at[dst_v.at[j - 1]], ssem[ns]).wait()
                pltpu.async_copy(
                    new_hbm.at[src_v.at[j + 1]], bufs[ns], gsem[ns])
            pltpu.make_async_copy(
                new_hbm.at[src_v.at[j]], bufs[s], gsem[s]).wait()
            pltpu.async_copy(bufs[s], mem_out.at[dst_v.at[j]], ssem[s])

        for j in range(ntc):
            pltpu.make_async_copy(
                ts_hbm.at[srcf_v.at[j]], ts_v.at[j], semt).wait()
        for j in range(ntc):
            pltpu.async_copy(ts_v.at[j], last_out.at[dstf_v.at[j]], semt)
        for j in range(ntc):
            pltpu.make_async_copy(
                ts_v.at[j], last_out.at[dstf_v.at[j]], semt).wait()

        pltpu.make_async_copy(
            bufs[(nch - 1) % 2], mem_out.at[dst_v.at[nch - 1]],
            ssem[(nch - 1) % 2]).wait()
        if nch >= 2:
            pltpu.make_async_copy(
                bufs[(nch - 2) % 2], mem_out.at[dst_v.at[nch - 2]],
                ssem[(nch - 2) % 2]).wait()

    k(new_mem, ts, dst3, src3, dstf, srcf, mem_ref, last_ref)


def _tc_copy(mem, last2d):
    u, d = mem.shape
    rows = 1000
    grid = u // rows
    lrows = last2d.shape[0] // grid
    lcols = last2d.shape[1]

    def body(m_r, l_r, mo_r, lo_r):
        mo_r[:] = m_r[:]
        lo_r[:] = l_r[:]

    return pl.pallas_call(
        body,
        grid=(grid,),
        in_specs=[
            pl.BlockSpec((rows, d), lambda i: (i, 0)),
            pl.BlockSpec((lrows, lcols), lambda i: (i, 0)),
        ],
        out_specs=[
            pl.BlockSpec((rows, d), lambda i: (i, 0)),
            pl.BlockSpec((lrows, lcols), lambda i: (i, 0)),
        ],
        out_shape=[
            jax.ShapeDtypeStruct((u, d), jnp.float32),
            jax.ShapeDtypeStruct(last2d.shape, jnp.float32),
        ],
        compiler_params=pltpu.CompilerParams(
            dimension_semantics=("arbitrary",)),
    )(mem, last2d)


def _dense(um, ie, ft, w1, b1, w2, b2, w_ih, w_hh, b_ih, b_hh):
    b, d = um.shape
    msg = w1.shape[0]
    tb = 1024
    grid = b // tb

    bf16 = jnp.bfloat16
    w1t = w1.T.astype(bf16)
    w1a, w1b, w1c = w1t[:d], w1t[d:2 * d], w1t[2 * d:]
    w2t = w2.T.astype(bf16)
    wih = w_ih.T.astype(bf16)
    whh = w_hh.T.astype(bf16)
    b1r = b1.reshape(1, -1)
    b2r = b2.reshape(1, -1)
    bihr = b_ih.reshape(1, -1)
    bhhr = b_hh.reshape(1, -1)

    def body(um_r, ie_r, ft_r, w1a_r, w1b_r, w1c_r, w2t_r, wih_r, whh_r,
             b1_r, b2_r, bih_r, bhh_r, out_r):
        umv = um_r[:]
        f32 = jnp.float32
        bf = jnp.bfloat16
        umb = umv.astype(bf)
        x = jnp.dot(umb, w1a_r[:], preferred_element_type=f32)
        x = x + jnp.dot(ie_r[:].astype(bf), w1b_r[:], preferred_element_type=f32)
        x = x + jnp.dot(ft_r[:].astype(bf), w1c_r[:], preferred_element_type=f32)
        h1 = jnp.maximum(x + b1_r[:], 0.0)
        m = jnp.dot(h1.astype(bf), w2t_r[:], preferred_element_type=f32) + b2_r[:]
        gi = jnp.dot(m.astype(bf), wih_r[:], preferred_element_type=f32) + bih_r[:]
        gh = jnp.dot(umb, whh_r[:], preferred_element_type=f32) + bhh_r[:]
        r = jax.nn.sigmoid(gi[:, :d] + gh[:, :d])
        z = jax.nn.sigmoid(gi[:, d:2 * d] + gh[:, d:2 * d])
        n = jnp.tanh(gi[:, 2 * d:] + r * gh[:, 2 * d:])
        out_r[:] = (1.0 - z) * n + z * umv

    const = lambda shape: pl.BlockSpec(shape, lambda i: (0, 0))
    batch = lambda shape: pl.BlockSpec(shape, lambda i: (i, 0))
    return pl.pallas_call(
        body,
        grid=(grid,),
        in_specs=[
            batch((tb, d)), batch((tb, d)), batch((tb, msg)),
            const((d, msg)), const((d, msg)), const((msg, msg)),
            const((msg, msg)), const((msg, 3 * d)), const((d, 3 * d)),
            const((1, msg)), const((1, msg)),
            const((1, 3 * d)), const((1, 3 * d)),
        ],
        out_specs=batch((tb, d)),
        out_shape=jax.ShapeDtypeStruct((b, d), jnp.float32),
        compiler_params=pltpu.CompilerParams(
            dimension_semantics=("arbitrary",)),
    )(um, ie, ft, w1a, w1b, w1c, w2t, wih, whh, b1r, b2r, bihr, bhhr)


def kernel(memory, last_update, user_indices, item_embedding,
           interaction_features, timestamps,
           w1, b1, w2, b2, w_ih, w_hh, b_ih, b_hh):
    u = memory.shape[0]
    b = user_indices.shape[0]
    nch = b // (NW * CH)

    ui = user_indices.astype(jnp.int32)
    iota = jnp.arange(b, dtype=jnp.int32)
    win = jnp.zeros((u,), jnp.int32).at[ui].set(iota)
    src = win[ui]

    idx3 = ui.reshape(NW, nch, CH)
    src3 = src.reshape(NW, nch, CH)
    ntc = b // (NW * 128)
    idxf = ui.reshape(NW, ntc, 128)
    srcf = src.reshape(NW, ntc, 128)

    user_memory = _sc_gather(memory, idx3, nch)
    new_memory = _dense(user_memory, item_embedding, interaction_features,
                        w1, b1, w2, b2, w_ih, w_hh, b_ih, b_hh)

    mem_ref = jax.new_ref(memory)
    last_ref = jax.new_ref(last_update)
    _sc_scatter(mem_ref, last_ref, new_memory, timestamps,
                idx3, src3, idxf, srcf, nch)
    return new_memory, mem_ref[...], last_ref[...]


# speedup vs baseline: 1.0385x; 1.0006x over previous
import functools

import jax
import jax.numpy as jnp
from jax import lax
from jax.experimental import pallas as pl
from jax.experimental.pallas import tpu as pltpu
from jax.experimental.pallas import tpu_sc as plsc

NC = 2
NS = 16
NW = NC * NS
CH = 64


def _mesh():
    return plsc.VectorSubcoreMesh(core_axis_name="c", subcore_axis_name="s")


def _sc_gather(mem, idx3, nch):
    d = mem.shape[1]
    b = NW * nch * CH

    @functools.partial(
        pl.kernel,
        mesh=_mesh(),
        out_type=jax.ShapeDtypeStruct((b, d), jnp.float32),
        scratch_types=[
            pltpu.VMEM((nch, CH), jnp.int32),
            pltpu.VMEM((CH, d), jnp.float32),
            pltpu.VMEM((CH, d), jnp.float32),
            pltpu.SemaphoreType.DMA,
            pltpu.SemaphoreType.DMA,
            pltpu.SemaphoreType.DMA,
            pltpu.SemaphoreType.DMA,
        ],
    )
    def k(mem_hbm, idx_hbm, out_hbm, idx_v, rows0, rows1, g0, g1, w0, w1):
        wid = lax.axis_index("s") * NC + lax.axis_index("c")
        base = wid * nch * CH
        pltpu.sync_copy(idx_hbm.at[wid], idx_v)
        bufs = (rows0, rows1)
        gsem = (g0, g1)
        wsem = (w0, w1)

        def wb_slice(j):
            return out_hbm.at[pl.ds(base + j * CH, CH)]

        pltpu.async_copy(mem_hbm.at[idx_v.at[0]], bufs[0], gsem[0])
        for j in range(nch):
            s = j % 2
            if j + 1 < nch:
                ns = (j + 1) % 2
                if j >= 1:
                    pltpu.make_async_copy(
                        bufs[ns], wb_slice(j - 1), wsem[ns]).wait()
                pltpu.async_copy(
                    mem_hbm.at[idx_v.at[j + 1]], bufs[ns], gsem[ns])
            pltpu.make_async_copy(
                mem_hbm.at[idx_v.at[j]], bufs[s], gsem[s]).wait()
            pltpu.async_copy(bufs[s], wb_slice(j), wsem[s])
        pltpu.make_async_copy(
            bufs[(nch - 1) % 2], wb_slice(nch - 1), wsem[(nch - 1) % 2]).wait()
        if nch >= 2:
            pltpu.make_async_copy(
                bufs[(nch - 2) % 2], wb_slice(nch - 2),
                wsem[(nch - 2) % 2]).wait()

    return k(mem, idx3)


def _sc_scatter(mem_ref, last_ref, new_mem, ts, dst3, src3, dstf, srcf, nch):
    d = new_mem.shape[1]
    ntc = dstf.shape[1]

    @functools.partial(
        pl.kernel,
        mesh=_mesh(),
        out_type=(),
        scratch_types=[
            pltpu.VMEM((nch, CH), jnp.int32),
            pltpu.VMEM((nch, CH), jnp.int32),
            pltpu.VMEM((ntc, 128), jnp.int32),
            pltpu.VMEM((ntc, 128), jnp.int32),
            pltpu.VMEM((CH, d), jnp.float32),
            pltpu.VMEM((CH, d), jnp.float32),
            pltpu.VMEM((ntc, 128), jnp.float32),
            pltpu.SemaphoreType.DMA,
            pltpu.SemaphoreType.DMA,
            pltpu.SemaphoreType.DMA,
            pltpu.SemaphoreType.DMA,
            pltpu.SemaphoreType.DMA,
        ],
    )
    def k(new_hbm, ts_hbm, dst_hbm, src_hbm, dstf_hbm, srcf_hbm,
          mem_out, last_out,
          dst_v, src_v, dstf_v, srcf_v, rows0, rows1, ts_v,
          g0, g1, s0, s1, semt):
        wid = lax.axis_index("s") * NC + lax.axis_index("c")
        pltpu.sync_copy(dst_hbm.at[wid], dst_v)
        pltpu.sync_copy(src_hbm.at[wid], src_v)
        pltpu.sync_copy(dstf_hbm.at[wid], dstf_v)
        pltpu.sync_copy(srcf_hbm.at[wid], srcf_v)

        for j in range(ntc):
            pltpu.async_copy(ts_hbm.at[srcf_v.at[j]], ts_v.at[j], semt)

        bufs = (rows0, rows1)
        gsem = (g0, g1)
        ssem = (s0, s1)
        pltpu.async_copy(new_hbm.at[src_v.at[0]], bufs[0], gsem[0])
        for j in range(nch):
            s = j % 2
            if j + 1 < nch:
                ns = (j + 1) % 2
                if j >= 1:
                    pltpu.make_async_copy(
                        bufs[ns], mem_out.at[dst_v.at[j - 1]], ssem[ns]).wait()
                pltpu.async_copy(
                    new_hbm.at[src_v.at[j + 1]], bufs[ns], gsem[ns])
            pltpu.make_async_copy(
                new_hbm.at[src_v.at[j]], bufs[s], gsem[s]).wait()
            pltpu.async_copy(bufs[s], mem_out.at[dst_v.at[j]], ssem[s])

        for j in range(ntc):
            pltpu.make_async_copy(
                ts_hbm.at[srcf_v.at[j]], ts_v.at[j], semt).wait()
        for j in range(ntc):
            pltpu.async_copy(ts_v.at[j], last_out.at[dstf_v.at[j]], semt)
        for j in range(ntc):
            pltpu.make_async_copy(
                ts_v.at[j], last_out.at[dstf_v.at[j]], semt).wait()

        pltpu.make_async_copy(
            bufs[(nch - 1) % 2], mem_out.at[dst_v.at[nch - 1]],
            ssem[(nch - 1) % 2]).wait()
        if nch >= 2:
            pltpu.make_async_copy(
                bufs[(nch - 2) % 2], mem_out.at[dst_v.at[nch - 2]],
                ssem[(nch - 2) % 2]).wait()

    k(new_mem, ts, dst3, src3, dstf, srcf, mem_ref, last_ref)


def _dense(um, ie, ft, w1, b1, w2, b2, w_ih, w_hh, b_ih, b_hh):
    b, d = um.shape
    msg = w1.shape[0]
    tb = 1024
    grid = b // tb

    bf16 = jnp.bfloat16
    w1t = w1.T.astype(bf16)
    w1a, w1b, w1c = w1t[:d], w1t[d:2 * d], w1t[2 * d:]
    w2t = w2.T.astype(bf16)
    wih = w_ih.T.astype(bf16)
    whh = w_hh.T.astype(bf16)
    b1r = b1.reshape(1, -1)
    b2r = b2.reshape(1, -1)
    bihr = b_ih.reshape(1, -1)
    bhhr = b_hh.reshape(1, -1)

    def body(um_r, ie_r, ft_r, w1a_r, w1b_r, w1c_r, w2t_r, wih_r, whh_r,
             b1_r, b2_r, bih_r, bhh_r, out_r):
        umv = um_r[:]
        f32 = jnp.float32
        bf = jnp.bfloat16
        umb = umv.astype(bf)
        x = jnp.dot(umb, w1a_r[:], preferred_element_type=f32)
        x = x + jnp.dot(ie_r[:].astype(bf), w1b_r[:], preferred_element_type=f32)
        x = x + jnp.dot(ft_r[:].astype(bf), w1c_r[:], preferred_element_type=f32)
        h1 = jnp.maximum(x + b1_r[:], 0.0)
        m = jnp.dot(h1.astype(bf), w2t_r[:], preferred_element_type=f32) + b2_r[:]
        gi = jnp.dot(m.astype(bf), wih_r[:], preferred_element_type=f32) + bih_r[:]
        gh = jnp.dot(umb, whh_r[:], preferred_element_type=f32) + bhh_r[:]
        r = jax.nn.sigmoid(gi[:, :d] + gh[:, :d])
        z = jax.nn.sigmoid(gi[:, d:2 * d] + gh[:, d:2 * d])
        n = jnp.tanh(gi[:, 2 * d:] + r * gh[:, 2 * d:])
        out_r[:] = (1.0 - z) * n + z * umv

    const = lambda shape: pl.BlockSpec(shape, lambda i: (0, 0))
    batch = lambda shape: pl.BlockSpec(shape, lambda i: (i, 0))
    return pl.pallas_call(
        body,
        grid=(grid,),
        in_specs=[
            batch((tb, d)), batch((tb, d)), batch((tb, msg)),
            const((d, msg)), const((d, msg)), const((msg, msg)),
            const((msg, msg)), const((msg, 3 * d)), const((d, 3 * d)),
            const((1, msg)), const((1, msg)),
            const((1, 3 * d)), const((1, 3 * d)),
        ],
        out_specs=batch((tb, d)),
        out_shape=jax.ShapeDtypeStruct((b, d), jnp.float32),
        compiler_params=pltpu.CompilerParams(
            dimension_semantics=("arbitrary",)),
    )(um, ie, ft, w1a, w1b, w1c, w2t, wih, whh, b1r, b2r, bihr, bhhr)


def kernel(memory, last_update, user_indices, item_embedding,
           interaction_features, timestamps,
           w1, b1, w2, b2, w_ih, w_hh, b_ih, b_hh):
    u = memory.shape[0]
    b = user_indices.shape[0]
    nch = b // (NW * CH)

    ui = user_indices.astype(jnp.int32)
    iota = jnp.arange(b, dtype=jnp.int32)
    win = jnp.zeros((u,), jnp.int32).at[ui].set(iota)
    src = win[ui]

    idx3 = ui.reshape(NW, nch, CH)
    src3 = src.reshape(NW, nch, CH)
    ntc = b // (NW * 128)
    idxf = ui.reshape(NW, ntc, 128)
    srcf = src.reshape(NW, ntc, 128)

    user_memory = _sc_gather(memory, idx3, nch)
    new_memory = _dense(user_memory, item_embedding, interaction_features,
                        w1, b1, w2, b2, w_ih, w_hh, b_ih, b_hh)

    mem_ref = jax.new_ref(memory)
    last_ref = jax.new_ref(last_update)
    _sc_scatter(mem_ref, last_ref, new_memory, timestamps,
                idx3, src3, idxf, srcf, nch)
    return new_memory, mem_ref[...], last_ref[...]
